# baseline (speedup 1.0000x reference)
import functools

import jax
import jax.numpy as jnp
from jax import lax
from jax.experimental import pallas as pl
from jax.experimental.pallas import tpu as pltpu
from jax.experimental.pallas import tpu_sc as plsc

NUM_ENTITIES = 1000000
EMBED_DIM = 32
BATCH = 16384

NC = 2
NS = 16
NW = NC * NS
BPW = BATCH // NW
HALF = BPW // 2
IDX_ROW = 128
N_IDX_ROWS = HALF * EMBED_DIM // IDX_ROW
LANES = 16


def _neg_sqrt(s):
    i = lax.bitcast_convert_type(s, jnp.int32)
    i = jnp.full((LANES,), 0x5F3759DF, jnp.int32) - (i >> 1)
    r = lax.bitcast_convert_type(i, jnp.float32)
    for _ in range(3):
        r = r * (1.5 - 0.5 * s * r * r)
    return -(s * r)


def _sc_body(batch_hbm, table_hbm, out_hbm,
             batch_v, idx_h, idx_t, rows_h, rows_t, out_v, sem):
    wid = lax.axis_index("s") * NC + lax.axis_index("c")
    base = wid * BPW

    pltpu.sync_copy(batch_hbm.at[pl.ds(base * 3, BPW * 3)], batch_v)

    iota = lax.iota(jnp.int32, LANES)

    for half in range(2):
        def build_body(g, carry):
            ri = (half * HALF + g * LANES + iota) * 3
            eh = plsc.load_gather(batch_v, [ri])
            et = plsc.load_gather(batch_v, [ri + 2])
            for d in range(EMBED_DIM):
                off = g * (LANES * EMBED_DIM) + d * LANES
                sl = pl.ds(off % IDX_ROW, LANES)
                idx_h[off // IDX_ROW, sl] = eh + d * NUM_ENTITIES
                idx_t[off // IDX_ROW, sl] = et + d * NUM_ENTITIES
            return carry

        lax.fori_loop(0, HALF // LANES, build_body, 0)

        copies = []
        for j in range(N_IDX_ROWS):
            copies.append(pltpu.make_async_copy(
                table_hbm.at[idx_h.at[j]],
                rows_h.at[pl.ds(j * IDX_ROW, IDX_ROW)], sem))
            copies.append(pltpu.make_async_copy(
                table_hbm.at[idx_t.at[j]],
                rows_t.at[pl.ds(j * IDX_ROW, IDX_ROW)], sem))
        for cp in copies:
            cp.start()
        for cp in copies:
            cp.wait()

        def chunk_body(g, carry):
            gbase = g * (LANES * EMBED_DIM)
            acc = jnp.zeros((LANES,), jnp.float32)
            for d in range(EMBED_DIM):
                hv = rows_h[pl.ds(gbase + d * LANES, LANES)]
                tv = rows_t[pl.ds(gbase + d * LANES, LANES)]
                df = hv - tv
                acc = acc + df * df
            out_v[pl.ds(half * HALF + g * LANES, LANES)] = (
                _neg_sqrt(acc + 1e-12))
            return carry

        lax.fori_loop(0, HALF // LANES, chunk_body, 0)

    pltpu.sync_copy(out_v, out_hbm.at[pl.ds(base, BPW)])


@functools.partial(jax.jit, static_argnames=())
def _sc_score(batch, entity_embeddings):
    mesh = plsc.VectorSubcoreMesh(core_axis_name="c", subcore_axis_name="s")
    call = pl.kernel(
        _sc_body,
        out_type=jax.ShapeDtypeStruct((BATCH,), jnp.float32),
        mesh=mesh,
        compiler_params=pltpu.CompilerParams(
            needs_layout_passes=False, use_tc_tiling_on_sc=False),
        scratch_types=[
            pltpu.VMEM((BPW * 3,), jnp.int32),
            pltpu.VMEM((N_IDX_ROWS, IDX_ROW), jnp.int32),
            pltpu.VMEM((N_IDX_ROWS, IDX_ROW), jnp.int32),
            pltpu.VMEM((HALF * EMBED_DIM,), jnp.float32),
            pltpu.VMEM((HALF * EMBED_DIM,), jnp.float32),
            pltpu.VMEM((BPW,), jnp.float32),
            pltpu.SemaphoreType.DMA,
        ],
    )
    return call(batch.reshape(-1), entity_embeddings.T.reshape(-1))


def kernel(batch, entity_embeddings):
    return _sc_score(batch, entity_embeddings)


# speedup vs baseline: 1.4388x; 1.4388x over previous
import functools

import jax
import jax.numpy as jnp
from jax import lax
from jax.experimental import pallas as pl
from jax.experimental.pallas import tpu as pltpu
from jax.experimental.pallas import tpu_sc as plsc

NUM_ENTITIES = 1000000
EMBED_DIM = 32
BATCH = 16384

NC = 2
NS = 16
NW = NC * NS
BPW = BATCH // NW
HALF = BPW // 2
IDX_ROW = 128
N_IDX_ROWS = HALF * EMBED_DIM // IDX_ROW
LANES = 16

N_FULL_TCOLS = NUM_ENTITIES // 128
TAIL_START = N_FULL_TCOLS * 128
TAIL = NUM_ENTITIES - TAIL_START
TCOLS_PER_W = 245
LAST_T0 = N_FULL_TCOLS - TCOLS_PER_W
CHUNK_E = TCOLS_PER_W * 128


def _linearize_body(table_t_hbm, tail_hbm, flat_hbm,
                    buf0, buf1, tail_v, rsem, wsem):
    wid = lax.axis_index("s") * NC + lax.axis_index("c")
    t0 = (wid * LAST_T0) // (NW - 1)
    e0 = pl.multiple_of(t0 * 128, 128)
    bufs = (buf0, buf1)

    def read_cp(d):
        return pltpu.make_async_copy(
            table_t_hbm.at[d, pl.ds(e0, CHUNK_E)], bufs[d % 2], rsem)

    def write_cp(d):
        return pltpu.make_async_copy(
            bufs[d % 2],
            flat_hbm.at[pl.ds(d * NUM_ENTITIES + e0, CHUNK_E)], wsem)

    read_cp(0).start()
    for d in range(EMBED_DIM):
        read_cp(d).wait()
        if d + 1 < EMBED_DIM:
            if d >= 1:
                write_cp(d - 1).wait()
            read_cp(d + 1).start()
        write_cp(d).start()
    write_cp(EMBED_DIM - 1).wait()

    @pl.when(wid == 0)
    def _():
        pltpu.sync_copy(tail_hbm, tail_v)
        for d in range(EMBED_DIM):
            pltpu.make_async_copy(
                tail_v.at[pl.ds(d * TAIL, TAIL)],
                flat_hbm.at[pl.ds(d * NUM_ENTITIES + TAIL_START, TAIL)],
                wsem).start()
        for d in range(EMBED_DIM):
            pltpu.make_async_copy(
                tail_v.at[pl.ds(d * TAIL, TAIL)],
                flat_hbm.at[pl.ds(d * NUM_ENTITIES + TAIL_START, TAIL)],
                wsem).wait()


def _neg_sqrt(s):
    i = lax.bitcast_convert_type(s, jnp.int32)
    i = jnp.full((LANES,), 0x5F3759DF, jnp.int32) - (i >> 1)
    r = lax.bitcast_convert_type(i, jnp.float32)
    for _ in range(3):
        r = r * (1.5 - 0.5 * s * r * r)
    return -(s * r)


def _sc_body(batch_hbm, table_hbm, out_hbm,
             batch_v, idx_h, idx_t, rows_h, rows_t, out_v, sem):
    wid = lax.axis_index("s") * NC + lax.axis_index("c")
    base = wid * BPW

    pltpu.sync_copy(batch_hbm.at[pl.ds(base * 3, BPW * 3)], batch_v)

    iota = lax.iota(jnp.int32, LANES)

    for half in range(2):
        def build_body(g, carry):
            ri = (half * HALF + g * LANES + iota) * 3
            eh = plsc.load_gather(batch_v, [ri])
            et = plsc.load_gather(batch_v, [ri + 2])
            for d in range(EMBED_DIM):
                off = g * (LANES * EMBED_DIM) + d * LANES
                sl = pl.ds(off % IDX_ROW, LANES)
                idx_h[off // IDX_ROW, sl] = eh + d * NUM_ENTITIES
                idx_t[off // IDX_ROW, sl] = et + d * NUM_ENTITIES
            return carry

        lax.fori_loop(0, HALF // LANES, build_body, 0)

        copies = []
        for j in range(N_IDX_ROWS):
            copies.append(pltpu.make_async_copy(
                table_hbm.at[idx_h.at[j]],
                rows_h.at[pl.ds(j * IDX_ROW, IDX_ROW)], sem))
            copies.append(pltpu.make_async_copy(
                table_hbm.at[idx_t.at[j]],
                rows_t.at[pl.ds(j * IDX_ROW, IDX_ROW)], sem))
        for cp in copies:
            cp.start()
        for cp in copies:
            cp.wait()

        def chunk_body(g, carry):
            gbase = g * (LANES * EMBED_DIM)
            acc = jnp.zeros((LANES,), jnp.float32)
            for d in range(EMBED_DIM):
                hv = rows_h[pl.ds(gbase + d * LANES, LANES)]
                tv = rows_t[pl.ds(gbase + d * LANES, LANES)]
                df = hv - tv
                acc = acc + df * df
            out_v[pl.ds(half * HALF + g * LANES, LANES)] = (
                _neg_sqrt(acc + 1e-12))
            return carry

        lax.fori_loop(0, HALF // LANES, chunk_body, 0)

    pltpu.sync_copy(out_v, out_hbm.at[pl.ds(base, BPW)])


@functools.partial(jax.jit, static_argnames=())
def _sc_score(batch, entity_embeddings):
    mesh = plsc.VectorSubcoreMesh(core_axis_name="c", subcore_axis_name="s")

    linearize = pl.kernel(
        _linearize_body,
        out_type=jax.ShapeDtypeStruct((NUM_ENTITIES * EMBED_DIM,),
                                      jnp.float32),
        mesh=mesh,
        compiler_params=pltpu.CompilerParams(
            needs_layout_passes=False, use_tc_tiling_on_sc=True),
        scratch_types=[pltpu.VMEM((CHUNK_E,), jnp.float32),
                       pltpu.VMEM((CHUNK_E,), jnp.float32),
                       pltpu.VMEM((TAIL * EMBED_DIM,), jnp.float32),
                       pltpu.SemaphoreType.DMA,
                       pltpu.SemaphoreType.DMA],
    )
    flat = jnp.concatenate(
        [entity_embeddings[:, d] for d in range(EMBED_DIM)])

    gather = pl.kernel(
        _sc_body,
        out_type=jax.ShapeDtypeStruct((BATCH,), jnp.float32),
        mesh=mesh,
        compiler_params=pltpu.CompilerParams(
            needs_layout_passes=False, use_tc_tiling_on_sc=False),
        scratch_types=[
            pltpu.VMEM((BPW * 3,), jnp.int32),
            pltpu.VMEM((N_IDX_ROWS, IDX_ROW), jnp.int32),
            pltpu.VMEM((N_IDX_ROWS, IDX_ROW), jnp.int32),
            pltpu.VMEM((HALF * EMBED_DIM,), jnp.float32),
            pltpu.VMEM((HALF * EMBED_DIM,), jnp.float32),
            pltpu.VMEM((BPW,), jnp.float32),
            pltpu.SemaphoreType.DMA,
        ],
    )
    return gather(batch.reshape(-1), flat)


def kernel(batch, entity_embeddings):
    return _sc_score(batch, entity_embeddings)


# speedup vs baseline: 12.7389x; 8.8538x over previous
import functools

import jax
import jax.numpy as jnp
from jax import lax
from jax.experimental import pallas as pl
from jax.experimental.pallas import tpu as pltpu
from jax.experimental.pallas import tpu_sc as plsc

NUM_ENTITIES = 1000000
EMBED_DIM = 32
BATCH = 16384

NC = 2
NS = 16
NW = NC * NS
BPW = BATCH // NW
HALF = BPW // 2
IDX_ROW = 128
N_IDX_ROWS = HALF * EMBED_DIM // IDX_ROW
LANES = 16

N_FULL_TCOLS = NUM_ENTITIES // 128
TAIL_START = N_FULL_TCOLS * 128
TAIL = NUM_ENTITIES - TAIL_START
TCOLS_PER_W = 245
LAST_T0 = N_FULL_TCOLS - TCOLS_PER_W
CHUNK_E = TCOLS_PER_W * 128


T_PER_W = 244
E_PER_W = T_PER_W * 128
X_START = 32 * T_PER_W * 128
X_LEN = TAIL_START - X_START


def _linearize_body(table_t_hbm, tail_hbm, flat_hbm,
                    buf0, buf1, tail_v, rsem, wsem):
    wid = lax.axis_index("s") * NC + lax.axis_index("c")
    e0 = pl.multiple_of(wid * E_PER_W, 128)

    for d in range(EMBED_DIM):
        buf = (buf0, buf1)[d % 2]
        pltpu.sync_copy(table_t_hbm.at[d, pl.ds(e0, E_PER_W)], buf)
        pltpu.sync_copy(
            buf, flat_hbm.at[pl.ds(d * NUM_ENTITIES + e0, E_PER_W)])

    @pl.when(wid == 0)
    def _():
        for d in range(EMBED_DIM):
            pltpu.sync_copy(
                table_t_hbm.at[d, pl.ds(X_START, X_LEN)],
                buf0.at[pl.ds(0, X_LEN)])
            pltpu.sync_copy(
                buf0.at[pl.ds(0, X_LEN)],
                flat_hbm.at[pl.ds(d * NUM_ENTITIES + X_START, X_LEN)])

    @pl.when(wid == 0)
    def _():
        pltpu.sync_copy(tail_hbm, tail_v)
        for d in range(EMBED_DIM):
            pltpu.make_async_copy(
                tail_v.at[pl.ds(d * TAIL, TAIL)],
                flat_hbm.at[pl.ds(d * NUM_ENTITIES + TAIL_START, TAIL)],
                wsem).start()
        for d in range(EMBED_DIM):
            pltpu.make_async_copy(
                tail_v.at[pl.ds(d * TAIL, TAIL)],
                flat_hbm.at[pl.ds(d * NUM_ENTITIES + TAIL_START, TAIL)],
                wsem).wait()


def _neg_sqrt(s):
    i = lax.bitcast_convert_type(s, jnp.int32)
    i = jnp.full((LANES,), 0x5F3759DF, jnp.int32) - (i >> 1)
    r = lax.bitcast_convert_type(i, jnp.float32)
    for _ in range(3):
        r = r * (1.5 - 0.5 * s * r * r)
    return -(s * r)


def _sc_body(batch_hbm, table_hbm, out_hbm,
             batch_v, idx_h, idx_t, rows_h, rows_t, out_v, sem):
    wid = lax.axis_index("s") * NC + lax.axis_index("c")
    base = wid * BPW

    pltpu.sync_copy(batch_hbm.at[pl.ds(base * 3, BPW * 3)], batch_v)

    iota = lax.iota(jnp.int32, LANES)

    for half in range(2):
        def build_body(g, carry):
            ri = (half * HALF + g * LANES + iota) * 3
            eh = plsc.load_gather(batch_v, [ri])
            et = plsc.load_gather(batch_v, [ri + 2])
            for d in range(EMBED_DIM):
                off = g * (LANES * EMBED_DIM) + d * LANES
                sl = pl.ds(off % IDX_ROW, LANES)
                idx_h[off // IDX_ROW, sl] = eh + d * NUM_ENTITIES
                idx_t[off // IDX_ROW, sl] = et + d * NUM_ENTITIES
            return carry

        lax.fori_loop(0, HALF // LANES, build_body, 0)

        copies = []
        for j in range(N_IDX_ROWS):
            copies.append(pltpu.make_async_copy(
                table_hbm.at[idx_h.at[j]],
                rows_h.at[pl.ds(j * IDX_ROW, IDX_ROW)], sem))
            copies.append(pltpu.make_async_copy(
                table_hbm.at[idx_t.at[j]],
                rows_t.at[pl.ds(j * IDX_ROW, IDX_ROW)], sem))
        for cp in copies:
            cp.start()
        for cp in copies:
            cp.wait()

        def chunk_body(g, carry):
            gbase = g * (LANES * EMBED_DIM)
            acc = jnp.zeros((LANES,), jnp.float32)
            for d in range(EMBED_DIM):
                hv = rows_h[pl.ds(gbase + d * LANES, LANES)]
                tv = rows_t[pl.ds(gbase + d * LANES, LANES)]
                df = hv - tv
                acc = acc + df * df
            out_v[pl.ds(half * HALF + g * LANES, LANES)] = (
                _neg_sqrt(acc + 1e-12))
            return carry

        lax.fori_loop(0, HALF // LANES, chunk_body, 0)

    pltpu.sync_copy(out_v, out_hbm.at[pl.ds(base, BPW)])


@functools.partial(jax.jit, static_argnames=())
def _sc_score(batch, entity_embeddings):
    mesh = plsc.VectorSubcoreMesh(core_axis_name="c", subcore_axis_name="s")

    linearize = pl.kernel(
        _linearize_body,
        out_type=jax.ShapeDtypeStruct((NUM_ENTITIES * EMBED_DIM,),
                                      jnp.float32),
        mesh=mesh,
        compiler_params=pltpu.CompilerParams(
            needs_layout_passes=False, use_tc_tiling_on_sc=True),
        scratch_types=[pltpu.VMEM((E_PER_W,), jnp.float32),
                       pltpu.VMEM((E_PER_W,), jnp.float32),
                       pltpu.VMEM((TAIL * EMBED_DIM,), jnp.float32),
                       pltpu.SemaphoreType.DMA,
                       pltpu.SemaphoreType.DMA],
    )
    tail_flat = entity_embeddings[TAIL_START:].T.reshape(-1)
    flat = linearize(entity_embeddings.T, tail_flat)

    gather = pl.kernel(
        _sc_body,
        out_type=jax.ShapeDtypeStruct((BATCH,), jnp.float32),
        mesh=mesh,
        compiler_params=pltpu.CompilerParams(
            needs_layout_passes=False, use_tc_tiling_on_sc=False),
        scratch_types=[
            pltpu.VMEM((BPW * 3,), jnp.int32),
            pltpu.VMEM((N_IDX_ROWS, IDX_ROW), jnp.int32),
            pltpu.VMEM((N_IDX_ROWS, IDX_ROW), jnp.int32),
            pltpu.VMEM((HALF * EMBED_DIM,), jnp.float32),
            pltpu.VMEM((HALF * EMBED_DIM,), jnp.float32),
            pltpu.VMEM((BPW,), jnp.float32),
            pltpu.SemaphoreType.DMA,
        ],
    )
    return gather(batch.reshape(-1), flat)


def kernel(batch, entity_embeddings):
    return _sc_score(batch, entity_embeddings)


# speedup vs baseline: 12.9528x; 1.0168x over previous
import functools

import jax
import jax.numpy as jnp
from jax import lax
from jax.experimental import pallas as pl
from jax.experimental.pallas import tpu as pltpu
from jax.experimental.pallas import tpu_sc as plsc

NUM_ENTITIES = 1000000
EMBED_DIM = 32
BATCH = 16384

NC = 2
NS = 16
NW = NC * NS
BPW = BATCH // NW
HALF = BPW // 2
IDX_ROW = 128
N_IDX_ROWS = HALF * EMBED_DIM // IDX_ROW
LANES = 16

N_FULL_TCOLS = NUM_ENTITIES // 128
TAIL_START = N_FULL_TCOLS * 128
TAIL = NUM_ENTITIES - TAIL_START
TCOLS_PER_W = 245
LAST_T0 = N_FULL_TCOLS - TCOLS_PER_W
CHUNK_E = TCOLS_PER_W * 128


T_PER_W = 244
E_PER_W = T_PER_W * 128
X_START = 32 * T_PER_W * 128
X_LEN = TAIL_START - X_START


def _linearize_body(table_t_hbm, tail_hbm, flat_hbm,
                    buf0, buf1, tail_v, rsem, wsem):
    wid = lax.axis_index("s") * NC + lax.axis_index("c")
    e0 = pl.multiple_of(wid * E_PER_W, 128)

    for d in range(EMBED_DIM):
        buf = (buf0, buf1)[d % 2]
        pltpu.sync_copy(table_t_hbm.at[d, pl.ds(e0, E_PER_W)], buf)
        pltpu.sync_copy(
            buf, flat_hbm.at[pl.ds(d * NUM_ENTITIES + e0, E_PER_W)])

    for w, xs in ((0, X_START), (1, X_START + X_LEN // 2)):
        @pl.when(wid == w)
        def _(xs=xs):
            for d in range(EMBED_DIM):
                pltpu.sync_copy(
                    table_t_hbm.at[d, pl.ds(xs, X_LEN // 2)],
                    buf0.at[pl.ds(0, X_LEN // 2)])
                pltpu.sync_copy(
                    buf0.at[pl.ds(0, X_LEN // 2)],
                    flat_hbm.at[pl.ds(d * NUM_ENTITIES + xs, X_LEN // 2)])

    @pl.when(wid == 2)
    def _():
        pltpu.sync_copy(tail_hbm, tail_v)
        for d in range(EMBED_DIM):
            pltpu.make_async_copy(
                tail_v.at[pl.ds(d * TAIL, TAIL)],
                flat_hbm.at[pl.ds(d * NUM_ENTITIES + TAIL_START, TAIL)],
                wsem).start()
        for d in range(EMBED_DIM):
            pltpu.make_async_copy(
                tail_v.at[pl.ds(d * TAIL, TAIL)],
                flat_hbm.at[pl.ds(d * NUM_ENTITIES + TAIL_START, TAIL)],
                wsem).wait()


def _neg_sqrt(s):
    i = lax.bitcast_convert_type(s, jnp.int32)
    i = jnp.full((LANES,), 0x5F3759DF, jnp.int32) - (i >> 1)
    r = lax.bitcast_convert_type(i, jnp.float32)
    for _ in range(3):
        r = r * (1.5 - 0.5 * s * r * r)
    return -(s * r)


def _sc_body(batch_hbm, table_hbm, out_hbm, batch_v,
             idx_h0, idx_t0, idx_h1, idx_t1,
             rows_h0, rows_t0, rows_h1, rows_t1, out_v, sem):
    wid = lax.axis_index("s") * NC + lax.axis_index("c")
    base = wid * BPW

    pltpu.sync_copy(batch_hbm.at[pl.ds(base * 3, BPW * 3)], batch_v)

    iota = lax.iota(jnp.int32, LANES)

    def build(half, idx_h, idx_t):
        def build_body(g, carry):
            ri = (half * HALF + g * LANES + iota) * 3
            eh = plsc.load_gather(batch_v, [ri])
            et = plsc.load_gather(batch_v, [ri + 2])
            for d in range(EMBED_DIM):
                off = g * (LANES * EMBED_DIM) + d * LANES
                sl = pl.ds(off % IDX_ROW, LANES)
                idx_h[off // IDX_ROW, sl] = eh + d * NUM_ENTITIES
                idx_t[off // IDX_ROW, sl] = et + d * NUM_ENTITIES
            return carry

        lax.fori_loop(0, HALF // LANES, build_body, 0)

    def make_copies(idx_h, idx_t, rows_h, rows_t):
        copies = []
        for j in range(N_IDX_ROWS):
            copies.append(pltpu.make_async_copy(
                table_hbm.at[idx_h.at[j]],
                rows_h.at[pl.ds(j * IDX_ROW, IDX_ROW)], sem))
            copies.append(pltpu.make_async_copy(
                table_hbm.at[idx_t.at[j]],
                rows_t.at[pl.ds(j * IDX_ROW, IDX_ROW)], sem))
        return copies

    def compute(half, rows_h, rows_t):
        def chunk_body(g, carry):
            gbase = g * (LANES * EMBED_DIM)
            acc = jnp.zeros((LANES,), jnp.float32)
            for d in range(EMBED_DIM):
                hv = rows_h[pl.ds(gbase + d * LANES, LANES)]
                tv = rows_t[pl.ds(gbase + d * LANES, LANES)]
                df = hv - tv
                acc = acc + df * df
            out_v[pl.ds(half * HALF + g * LANES, LANES)] = (
                _neg_sqrt(acc + 1e-12))
            return carry

        lax.fori_loop(0, HALF // LANES, chunk_body, 0)

    build(0, idx_h0, idx_t0)
    copies0 = make_copies(idx_h0, idx_t0, rows_h0, rows_t0)
    for cp in copies0:
        cp.start()
    build(1, idx_h1, idx_t1)
    for cp in copies0:
        cp.wait()
    copies1 = make_copies(idx_h1, idx_t1, rows_h1, rows_t1)
    for cp in copies1:
        cp.start()
    compute(0, rows_h0, rows_t0)
    for cp in copies1:
        cp.wait()
    compute(1, rows_h1, rows_t1)

    pltpu.sync_copy(out_v, out_hbm.at[pl.ds(base, BPW)])


@functools.partial(jax.jit, static_argnames=())
def _sc_score(batch, entity_embeddings):
    mesh = plsc.VectorSubcoreMesh(core_axis_name="c", subcore_axis_name="s")

    linearize = pl.kernel(
        _linearize_body,
        out_type=jax.ShapeDtypeStruct((NUM_ENTITIES * EMBED_DIM,),
                                      jnp.float32),
        mesh=mesh,
        compiler_params=pltpu.CompilerParams(
            needs_layout_passes=False, use_tc_tiling_on_sc=True),
        scratch_types=[pltpu.VMEM((E_PER_W,), jnp.float32),
                       pltpu.VMEM((E_PER_W,), jnp.float32),
                       pltpu.VMEM((TAIL * EMBED_DIM,), jnp.float32),
                       pltpu.SemaphoreType.DMA,
                       pltpu.SemaphoreType.DMA],
    )
    tail_flat = entity_embeddings[TAIL_START:].T.reshape(-1)
    flat = linearize(entity_embeddings.T, tail_flat)

    gather = pl.kernel(
        _sc_body,
        out_type=jax.ShapeDtypeStruct((BATCH,), jnp.float32),
        mesh=mesh,
        compiler_params=pltpu.CompilerParams(
            needs_layout_passes=False, use_tc_tiling_on_sc=False),
        scratch_types=[
            pltpu.VMEM((BPW * 3,), jnp.int32),
            pltpu.VMEM((N_IDX_ROWS, IDX_ROW), jnp.int32),
            pltpu.VMEM((N_IDX_ROWS, IDX_ROW), jnp.int32),
            pltpu.VMEM((N_IDX_ROWS, IDX_ROW), jnp.int32),
            pltpu.VMEM((N_IDX_ROWS, IDX_ROW), jnp.int32),
            pltpu.VMEM((HALF * EMBED_DIM,), jnp.float32),
            pltpu.VMEM((HALF * EMBED_DIM,), jnp.float32),
            pltpu.VMEM((HALF * EMBED_DIM,), jnp.float32),
            pltpu.VMEM((HALF * EMBED_DIM,), jnp.float32),
            pltpu.VMEM((BPW,), jnp.float32),
            pltpu.SemaphoreType.DMA,
        ],
    )
    return gather(batch.reshape(-1), flat)


def kernel(batch, entity_embeddings):
    return _sc_score(batch, entity_embeddings)
